# baseline (speedup 1.0000x reference)
import functools

import jax
import jax.numpy as jnp
from jax.experimental import pallas as pl
from jax.experimental.pallas import tpu as pltpu


def _rpn_level_kernel(x_ref, wt_ref, hw_ref, cb_ref, hb_ref, out_ref,
                      *, Wp, NB, T):
    cb = cb_ref[...]
    hb = hb_ref[...]

    def tile(i, carry):
        j0 = i * NB
        acc = jnp.zeros((256, NB), jnp.float32)
        for k in range(9):
            off = (k // 3) * Wp + (k % 3)
            base, r = (off // 128) * 128, off % 128
            if r == 0:
                rhs = x_ref[:, pl.ds(j0 + base, NB)]
            else:
                chunk = x_ref[:, pl.ds(j0 + base, NB + 128)]
                rhs = pltpu.roll(chunk, NB + 128 - r, axis=1)[:, :NB]
            acc = acc + jax.lax.dot_general(
                wt_ref[k], rhs, (((1,), (0,)), ((), ())),
                preferred_element_type=jnp.float32)
        t = jnp.maximum(acc + cb, 0.0).astype(jnp.bfloat16)
        o = jax.lax.dot_general(
            hw_ref[...], t, (((1,), (0,)), ((), ())),
            preferred_element_type=jnp.float32) + hb
        out_ref[:, pl.ds(j0, NB)] = o
        return carry

    jax.lax.fori_loop(0, T, tile, 0)


def _run_level(x, wt, hw, cb, hb, NB):
    N, C, H, W = x.shape
    Wp = W + 2
    Lr = H * Wp
    T = -(-Lr // NB)
    Lout = T * NB
    need = Lout + 2 * Wp + 2 + 128
    extra_rows = max(0, -(-(need - (H + 2) * Wp) // Wp))
    Ltot = (H + 2 + extra_rows) * Wp
    xp = jnp.pad(x.astype(jnp.bfloat16),
                 ((0, 0), (0, 0), (1, 1 + extra_rows), (1, 1)))
    xp = xp.reshape(N, C, Ltot)
    out = pl.pallas_call(
        functools.partial(_rpn_level_kernel, Wp=Wp, NB=NB, T=T),
        grid=(N,),
        in_specs=[
            pl.BlockSpec((None, C, Ltot), lambda b: (b, 0, 0)),
            pl.BlockSpec((9, C, C), lambda b: (0, 0, 0)),
            pl.BlockSpec((16, C), lambda b: (0, 0)),
            pl.BlockSpec((C, 1), lambda b: (0, 0)),
            pl.BlockSpec((16, 1), lambda b: (0, 0)),
        ],
        out_specs=pl.BlockSpec((None, 16, Lout), lambda b: (b, 0, 0)),
        out_shape=jax.ShapeDtypeStruct((N, 16, Lout), jnp.float32),
        compiler_params=pltpu.CompilerParams(
            dimension_semantics=("parallel",)),
    )(xp, wt, hw, cb, hb)
    o = out[:, :, :Lr].reshape(N, 16, H, Wp)[:, :, :, :W]
    return o[:, :3], o[:, 3:15]


_LEVEL_NB = (640, 384, 128, 384, 128)


def kernel(feature0, feature1, feature2, feature3, feature4,
           conv_w, conv_b, cls_w, cls_b, bbox_w, bbox_b):
    wt = conv_w.transpose(2, 3, 0, 1).reshape(9, 256, 256).astype(jnp.bfloat16)
    hw = jnp.concatenate(
        [cls_w[:, :, 0, 0], bbox_w[:, :, 0, 0],
         jnp.zeros((1, 256), cls_w.dtype)]).astype(jnp.bfloat16)
    cb = conv_b.reshape(256, 1)
    hb = jnp.concatenate(
        [cls_b, bbox_b, jnp.zeros((1,), cls_b.dtype)]).reshape(16, 1)
    logits, bbox = [], []
    for f, nb in zip((feature0, feature1, feature2, feature3, feature4),
                     _LEVEL_NB):
        lo, bb = _run_level(f, wt, hw, cb, hb, nb)
        logits.append(lo)
        bbox.append(bb)
    return tuple(logits) + tuple(bbox)
